# baseline (device time: 325416 ns/iter reference)
import jax
import jax.numpy as jnp
from jax import lax
from jax.experimental import pallas as pl
from jax.experimental.pallas import tpu as pltpu

NZ = 4
M = 4096
N = 4096
CHUNK = N // NZ
HALF = CHUNK // 2


def kernel(x):
    x = x.reshape(M, N).astype(jnp.bfloat16)

    def body(x_ref, out_ref, comm_r, comm_l, p_r, p_l,
             ss_r, rs_r, ss_l, rs_l, local_sems):
        xi = lax.axis_index("x")
        yi = lax.axis_index("y")
        zi = lax.axis_index("z")
        right = (zi + 1) % NZ
        left = (zi - 1) % NZ

        barrier_sem = pltpu.get_barrier_semaphore()
        for nbr in (left, right):
            pl.semaphore_signal(
                barrier_sem, inc=1,
                device_id=(xi, yi, nbr),
                device_id_type=pl.DeviceIdType.MESH,
            )
        pl.semaphore_wait(barrier_sem, 2)

        r_c0 = (zi + 3) % NZ
        r_c1 = (zi + 2) % NZ
        r_c2 = (zi + 1) % NZ
        l_c0 = (zi + 1) % NZ
        l_c1 = (zi + 2) % NZ
        l_c2 = (zi + 3) % NZ

        def lo(c):
            return pl.ds(c * CHUNK, HALF)

        def hi(c):
            return pl.ds(c * CHUNK + HALF, HALF)

        lds = [
            pltpu.make_async_copy(x_ref.at[:, lo(r_c1)], p_r.at[0],
                                  local_sems.at[0]),
            pltpu.make_async_copy(x_ref.at[:, lo(r_c2)], p_r.at[1],
                                  local_sems.at[1]),
            pltpu.make_async_copy(x_ref.at[:, hi(l_c1)], p_l.at[0],
                                  local_sems.at[2]),
            pltpu.make_async_copy(x_ref.at[:, hi(l_c2)], p_l.at[1],
                                  local_sems.at[3]),
            pltpu.make_async_copy(x_ref.at[:, lo(zi)],
                                  out_ref.at[:, pl.ds(0, HALF)],
                                  local_sems.at[4]),
            pltpu.make_async_copy(x_ref.at[:, hi(zi)],
                                  out_ref.at[:, pl.ds(HALF, HALF)],
                                  local_sems.at[5]),
        ]
        for ld in lds:
            ld.start()

        rdma_r0 = pltpu.make_async_remote_copy(
            src_ref=x_ref.at[:, lo(r_c0)], dst_ref=comm_r.at[0],
            send_sem=ss_r.at[0], recv_sem=rs_r.at[0],
            device_id=(xi, yi, right), device_id_type=pl.DeviceIdType.MESH)
        rdma_l0 = pltpu.make_async_remote_copy(
            src_ref=x_ref.at[:, hi(l_c0)], dst_ref=comm_l.at[0],
            send_sem=ss_l.at[0], recv_sem=rs_l.at[0],
            device_id=(xi, yi, left), device_id_type=pl.DeviceIdType.MESH)
        rdma_r0.start()
        rdma_l0.start()
        rdma_r0.wait()
        rdma_l0.wait()

        lds[0].wait()
        lds[2].wait()
        p_r[0] = p_r[0] + comm_r[0]
        p_l[0] = p_l[0] + comm_l[0]

        rdma_r1 = pltpu.make_async_remote_copy(
            src_ref=p_r.at[0], dst_ref=comm_r.at[1],
            send_sem=ss_r.at[1], recv_sem=rs_r.at[1],
            device_id=(xi, yi, right), device_id_type=pl.DeviceIdType.MESH)
        rdma_l1 = pltpu.make_async_remote_copy(
            src_ref=p_l.at[0], dst_ref=comm_l.at[1],
            send_sem=ss_l.at[1], recv_sem=rs_l.at[1],
            device_id=(xi, yi, left), device_id_type=pl.DeviceIdType.MESH)
        rdma_r1.start()
        rdma_l1.start()
        rdma_r1.wait()
        rdma_l1.wait()

        lds[1].wait()
        lds[3].wait()
        p_r[1] = p_r[1] + comm_r[1]
        p_l[1] = p_l[1] + comm_l[1]

        rdma_r2 = pltpu.make_async_remote_copy(
            src_ref=p_r.at[1], dst_ref=comm_r.at[2],
            send_sem=ss_r.at[2], recv_sem=rs_r.at[2],
            device_id=(xi, yi, right), device_id_type=pl.DeviceIdType.MESH)
        rdma_l2 = pltpu.make_async_remote_copy(
            src_ref=p_l.at[1], dst_ref=comm_l.at[2],
            send_sem=ss_l.at[2], recv_sem=rs_l.at[2],
            device_id=(xi, yi, left), device_id_type=pl.DeviceIdType.MESH)
        rdma_r2.start()
        rdma_l2.start()
        rdma_r2.wait()
        rdma_l2.wait()

        lds[4].wait()
        lds[5].wait()
        out_ref[:, pl.ds(0, HALF)] = out_ref[:, pl.ds(0, HALF)] + comm_r[2]
        out_ref[:, pl.ds(HALF, HALF)] = (
            out_ref[:, pl.ds(HALF, HALF)] + comm_l[2])

    return pl.pallas_call(
        body,
        out_shape=jax.ShapeDtypeStruct((M, CHUNK), jnp.bfloat16),
        in_specs=[pl.BlockSpec(memory_space=pltpu.MemorySpace.HBM)],
        out_specs=pl.BlockSpec(memory_space=pltpu.VMEM),
        scratch_shapes=[
            pltpu.VMEM((3, M, HALF), jnp.bfloat16),
            pltpu.VMEM((3, M, HALF), jnp.bfloat16),
            pltpu.VMEM((2, M, HALF), jnp.bfloat16),
            pltpu.VMEM((2, M, HALF), jnp.bfloat16),
            pltpu.SemaphoreType.DMA((3,)),
            pltpu.SemaphoreType.DMA((3,)),
            pltpu.SemaphoreType.DMA((3,)),
            pltpu.SemaphoreType.DMA((3,)),
            pltpu.SemaphoreType.DMA((6,)),
        ],
        compiler_params=pltpu.CompilerParams(
            collective_id=0,
            vmem_limit_bytes=100 * 1024 * 1024,
        ),
    )(x)


# device time: 298470 ns/iter; 1.0903x vs baseline; 1.0903x over previous
import jax
import jax.numpy as jnp
from jax import lax
from jax.experimental import pallas as pl
from jax.experimental.pallas import tpu as pltpu

NZ = 4
M = 4096
N = 4096
CHUNK = N // NZ
HALF = CHUNK // 2

BF = jnp.bfloat16
F32 = jnp.float32


def kernel(x):

    def body(x_ref, out_ref, comm_r, comm_l, s0_r, s0_l, fa,
             ss_r, rs_r, ss_l, rs_l, local_sems):
        xi = lax.axis_index("x")
        yi = lax.axis_index("y")
        zi = lax.axis_index("z")
        right = (zi + 1) % NZ
        left = (zi - 1) % NZ

        r_c0 = (zi + 3) % NZ
        r_c1 = (zi + 2) % NZ
        r_c2 = (zi + 1) % NZ
        l_c0 = (zi + 1) % NZ
        l_c1 = (zi + 2) % NZ
        l_c2 = (zi + 3) % NZ

        def lo(c):
            return pl.ds(c * CHUNK, HALF)

        def hi(c):
            return pl.ds(c * CHUNK + HALF, HALF)

        def load(col_slice, dst, sem_i):
            cp = pltpu.make_async_copy(
                x_ref.at[0, :, col_slice], dst, local_sems.at[sem_i])
            cp.start()
            return cp

        def ring_rdma(step, src_r, src_l):
            r = pltpu.make_async_remote_copy(
                src_ref=src_r, dst_ref=comm_r.at[step],
                send_sem=ss_r.at[step], recv_sem=rs_r.at[step],
                device_id=(xi, yi, right),
                device_id_type=pl.DeviceIdType.MESH)
            l = pltpu.make_async_remote_copy(
                src_ref=src_l, dst_ref=comm_l.at[step],
                send_sem=ss_l.at[step], recv_sem=rs_l.at[step],
                device_id=(xi, yi, left),
                device_id_type=pl.DeviceIdType.MESH)
            r.start()
            l.start()
            return r, l

        ld_r = load(lo(r_c0), fa.at[0], 0)
        ld_l = load(hi(l_c0), fa.at[1], 1)

        barrier_sem = pltpu.get_barrier_semaphore()
        for nbr in (left, right):
            pl.semaphore_signal(
                barrier_sem, inc=1,
                device_id=(xi, yi, nbr),
                device_id_type=pl.DeviceIdType.MESH,
            )
        pl.semaphore_wait(barrier_sem, 2)

        ld_r.wait()
        s0_r[...] = fa[0].astype(BF)
        ld_l.wait()
        s0_l[...] = fa[1].astype(BF)

        rdma_r, rdma_l = ring_rdma(0, s0_r, s0_l)
        ld_r = load(lo(r_c1), fa.at[0], 2)
        ld_l = load(hi(l_c1), fa.at[1], 3)
        rdma_r.wait()
        rdma_l.wait()

        ld_r.wait()
        ld_l.wait()
        comm_r[0] = (fa[0] + comm_r[0].astype(F32)).astype(BF)
        comm_l[0] = (fa[1] + comm_l[0].astype(F32)).astype(BF)

        rdma_r, rdma_l = ring_rdma(1, comm_r.at[0], comm_l.at[0])
        ld_r = load(lo(r_c2), fa.at[0], 4)
        ld_l = load(hi(l_c2), fa.at[1], 5)
        rdma_r.wait()
        rdma_l.wait()

        ld_r.wait()
        ld_l.wait()
        comm_r[1] = (fa[0] + comm_r[1].astype(F32)).astype(BF)
        comm_l[1] = (fa[1] + comm_l[1].astype(F32)).astype(BF)

        rdma_r, rdma_l = ring_rdma(2, comm_r.at[1], comm_l.at[1])
        ld_r = load(lo(zi), fa.at[0], 6)
        ld_l = load(hi(zi), fa.at[1], 7)
        rdma_r.wait()
        rdma_l.wait()

        ld_r.wait()
        ld_l.wait()
        out_ref[:, pl.ds(0, HALF)] = (
            fa[0] + comm_r[2].astype(F32)).astype(BF)
        out_ref[:, pl.ds(HALF, HALF)] = (
            fa[1] + comm_l[2].astype(F32)).astype(BF)

    return pl.pallas_call(
        body,
        out_shape=jax.ShapeDtypeStruct((M, CHUNK), BF),
        in_specs=[pl.BlockSpec(memory_space=pltpu.MemorySpace.HBM)],
        out_specs=pl.BlockSpec(memory_space=pltpu.VMEM),
        scratch_shapes=[
            pltpu.VMEM((3, M, HALF), BF),
            pltpu.VMEM((3, M, HALF), BF),
            pltpu.VMEM((M, HALF), BF),
            pltpu.VMEM((M, HALF), BF),
            pltpu.VMEM((2, M, HALF), F32),
            pltpu.SemaphoreType.DMA((3,)),
            pltpu.SemaphoreType.DMA((3,)),
            pltpu.SemaphoreType.DMA((3,)),
            pltpu.SemaphoreType.DMA((3,)),
            pltpu.SemaphoreType.DMA((8,)),
        ],
        compiler_params=pltpu.CompilerParams(
            collective_id=0,
            vmem_limit_bytes=62 * 1024 * 1024,
        ),
    )(x)


# device time: 290837 ns/iter; 1.1189x vs baseline; 1.0262x over previous
import jax
import jax.numpy as jnp
from jax import lax
from jax.experimental import pallas as pl
from jax.experimental.pallas import tpu as pltpu

NZ = 4
M = 4096
N = 4096
CHUNK = N // NZ
HALF = CHUNK // 2
NB = 2
RB = M // NB

BF = jnp.bfloat16
F32 = jnp.float32


def kernel(x):

    def body(x_ref, out_ref, comm_r, comm_l, s0_r, s0_l, fa,
             ss_r, rs_r, ss_l, rs_l, local_sems):
        xi = lax.axis_index("x")
        yi = lax.axis_index("y")
        zi = lax.axis_index("z")
        right = (zi + 1) % NZ
        left = (zi - 1) % NZ

        r_c0 = (zi + 3) % NZ
        r_adds = [(zi + 2) % NZ, (zi + 1) % NZ, zi]
        l_c0 = (zi + 1) % NZ
        l_adds = [(zi + 2) % NZ, (zi + 3) % NZ, zi]

        def lo(c):
            return pl.ds(c * CHUNK, HALF)

        def hi(c):
            return pl.ds(c * CHUNK + HALF, HALF)

        def load(col_slice, dst, sem_i):
            cp = pltpu.make_async_copy(
                x_ref.at[0, :, col_slice], dst, local_sems.at[sem_i])
            cp.start()
            return cp

        def rb(b):
            return slice(b * RB, (b + 1) * RB)

        def hop(comm, ss, rs, dev, s, b, src):
            return pltpu.make_async_remote_copy(
                src_ref=src,
                dst_ref=comm.at[s, pl.ds(b * RB, RB), :],
                send_sem=ss.at[s, b], recv_sem=rs.at[s, b],
                device_id=(xi, yi, dev),
                device_id_type=pl.DeviceIdType.MESH)

        ld_r = load(lo(r_c0), fa.at[0], 0)
        ld_l = load(hi(l_c0), fa.at[1], 1)

        barrier_sem = pltpu.get_barrier_semaphore()
        for nbr in (left, right):
            pl.semaphore_signal(
                barrier_sem, inc=1,
                device_id=(xi, yi, nbr),
                device_id_type=pl.DeviceIdType.MESH,
            )
        pl.semaphore_wait(barrier_sem, 2)

        sends = []

        ld_r.wait()
        ld_l.wait()
        for b in range(NB):
            s0_r[rb(b), :] = fa[0, rb(b), :].astype(BF)
            d = hop(comm_r, ss_r, rs_r, right, 0, b,
                    s0_r.at[pl.ds(b * RB, RB), :])
            d.start()
            sends.append(d)
            s0_l[rb(b), :] = fa[1, rb(b), :].astype(BF)
            d = hop(comm_l, ss_l, rs_l, left, 0, b,
                    s0_l.at[pl.ds(b * RB, RB), :])
            d.start()
            sends.append(d)

        ld_r = load(lo(r_adds[0]), fa.at[0], 2)
        ld_l = load(hi(l_adds[0]), fa.at[1], 3)

        for s in range(3):
            ld_r.wait()
            ld_l.wait()
            for b in range(NB):
                hop(comm_r, ss_r, rs_r, right, s, b,
                    s0_r.at[pl.ds(b * RB, RB), :]).wait_recv()
                hop(comm_l, ss_l, rs_l, left, s, b,
                    s0_l.at[pl.ds(b * RB, RB), :]).wait_recv()
                if s < 2:
                    comm_r[s, rb(b), :] = (
                        fa[0, rb(b), :]
                        + comm_r[s, rb(b), :].astype(F32)).astype(BF)
                    d = hop(comm_r, ss_r, rs_r, right, s + 1, b,
                            comm_r.at[s, pl.ds(b * RB, RB), :])
                    d.start()
                    sends.append(d)
                    comm_l[s, rb(b), :] = (
                        fa[1, rb(b), :]
                        + comm_l[s, rb(b), :].astype(F32)).astype(BF)
                    d = hop(comm_l, ss_l, rs_l, left, s + 1, b,
                            comm_l.at[s, pl.ds(b * RB, RB), :])
                    d.start()
                    sends.append(d)
                else:
                    out_ref[rb(b), pl.ds(0, HALF)] = (
                        fa[0, rb(b), :]
                        + comm_r[2, rb(b), :].astype(F32)).astype(BF)
                    out_ref[rb(b), pl.ds(HALF, HALF)] = (
                        fa[1, rb(b), :]
                        + comm_l[2, rb(b), :].astype(F32)).astype(BF)
            if s < 2:
                ld_r = load(lo(r_adds[s + 1]), fa.at[0], 4 + 2 * s)
                ld_l = load(hi(l_adds[s + 1]), fa.at[1], 5 + 2 * s)

        for d in sends:
            d.wait_send()

    return pl.pallas_call(
        body,
        out_shape=jax.ShapeDtypeStruct((M, CHUNK), BF),
        in_specs=[pl.BlockSpec(memory_space=pltpu.MemorySpace.HBM)],
        out_specs=pl.BlockSpec(memory_space=pltpu.VMEM),
        scratch_shapes=[
            pltpu.VMEM((3, M, HALF), BF),
            pltpu.VMEM((3, M, HALF), BF),
            pltpu.VMEM((M, HALF), BF),
            pltpu.VMEM((M, HALF), BF),
            pltpu.VMEM((2, M, HALF), F32),
            pltpu.SemaphoreType.DMA((3, NB)),
            pltpu.SemaphoreType.DMA((3, NB)),
            pltpu.SemaphoreType.DMA((3, NB)),
            pltpu.SemaphoreType.DMA((3, NB)),
            pltpu.SemaphoreType.DMA((8,)),
        ],
        compiler_params=pltpu.CompilerParams(
            collective_id=0,
            vmem_limit_bytes=62 * 1024 * 1024,
        ),
    )(x)


# device time: 290366 ns/iter; 1.1207x vs baseline; 1.0016x over previous
import jax
import jax.numpy as jnp
from jax import lax
from jax.experimental import pallas as pl
from jax.experimental.pallas import tpu as pltpu

NZ = 4
M = 4096
N = 4096
CHUNK = N // NZ
HALF = CHUNK // 2
NB = 4
RB = M // NB

BF = jnp.bfloat16
F32 = jnp.float32


def kernel(x):

    def body(x_ref, out_ref, comm_r, comm_l, s0_r, s0_l, fa,
             ss_r, rs_r, ss_l, rs_l, local_sems):
        xi = lax.axis_index("x")
        yi = lax.axis_index("y")
        zi = lax.axis_index("z")
        right = (zi + 1) % NZ
        left = (zi - 1) % NZ

        r_c0 = (zi + 3) % NZ
        r_adds = [(zi + 2) % NZ, (zi + 1) % NZ, zi]
        l_c0 = (zi + 1) % NZ
        l_adds = [(zi + 2) % NZ, (zi + 3) % NZ, zi]

        def lo(c):
            return pl.ds(c * CHUNK, HALF)

        def hi(c):
            return pl.ds(c * CHUNK + HALF, HALF)

        def load(col_slice, dst, sem_i):
            cp = pltpu.make_async_copy(
                x_ref.at[0, :, col_slice], dst, local_sems.at[sem_i])
            cp.start()
            return cp

        def rb(b):
            return slice(b * RB, (b + 1) * RB)

        def hop(comm, ss, rs, dev, s, b, src):
            return pltpu.make_async_remote_copy(
                src_ref=src,
                dst_ref=comm.at[s, pl.ds(b * RB, RB), :],
                send_sem=ss.at[s, b], recv_sem=rs.at[s, b],
                device_id=(xi, yi, dev),
                device_id_type=pl.DeviceIdType.MESH)

        ld_r = load(lo(r_c0), fa.at[0], 0)
        ld_l = load(hi(l_c0), fa.at[1], 1)

        barrier_sem = pltpu.get_barrier_semaphore()
        for nbr in (left, right):
            pl.semaphore_signal(
                barrier_sem, inc=1,
                device_id=(xi, yi, nbr),
                device_id_type=pl.DeviceIdType.MESH,
            )
        pl.semaphore_wait(barrier_sem, 2)

        sends = []

        ld_r.wait()
        ld_l.wait()
        for b in range(NB):
            s0_r[rb(b), :] = fa[0, rb(b), :].astype(BF)
            d = hop(comm_r, ss_r, rs_r, right, 0, b,
                    s0_r.at[pl.ds(b * RB, RB), :])
            d.start()
            sends.append(d)
            s0_l[rb(b), :] = fa[1, rb(b), :].astype(BF)
            d = hop(comm_l, ss_l, rs_l, left, 0, b,
                    s0_l.at[pl.ds(b * RB, RB), :])
            d.start()
            sends.append(d)

        ld_r = load(lo(r_adds[0]), fa.at[0], 2)
        ld_l = load(hi(l_adds[0]), fa.at[1], 3)

        for s in range(3):
            ld_r.wait()
            ld_l.wait()
            for b in range(NB):
                hop(comm_r, ss_r, rs_r, right, s, b,
                    s0_r.at[pl.ds(b * RB, RB), :]).wait_recv()
                hop(comm_l, ss_l, rs_l, left, s, b,
                    s0_l.at[pl.ds(b * RB, RB), :]).wait_recv()
                if s < 2:
                    comm_r[s, rb(b), :] = (
                        fa[0, rb(b), :]
                        + comm_r[s, rb(b), :].astype(F32)).astype(BF)
                    d = hop(comm_r, ss_r, rs_r, right, s + 1, b,
                            comm_r.at[s, pl.ds(b * RB, RB), :])
                    d.start()
                    sends.append(d)
                    comm_l[s, rb(b), :] = (
                        fa[1, rb(b), :]
                        + comm_l[s, rb(b), :].astype(F32)).astype(BF)
                    d = hop(comm_l, ss_l, rs_l, left, s + 1, b,
                            comm_l.at[s, pl.ds(b * RB, RB), :])
                    d.start()
                    sends.append(d)
                else:
                    out_ref[rb(b), pl.ds(0, HALF)] = (
                        fa[0, rb(b), :]
                        + comm_r[2, rb(b), :].astype(F32)).astype(BF)
                    out_ref[rb(b), pl.ds(HALF, HALF)] = (
                        fa[1, rb(b), :]
                        + comm_l[2, rb(b), :].astype(F32)).astype(BF)
            if s < 2:
                ld_r = load(lo(r_adds[s + 1]), fa.at[0], 4 + 2 * s)
                ld_l = load(hi(l_adds[s + 1]), fa.at[1], 5 + 2 * s)

        for d in sends:
            d.wait_send()

    return pl.pallas_call(
        body,
        out_shape=jax.ShapeDtypeStruct((M, CHUNK), BF),
        in_specs=[pl.BlockSpec(memory_space=pltpu.MemorySpace.HBM)],
        out_specs=pl.BlockSpec(memory_space=pltpu.VMEM),
        scratch_shapes=[
            pltpu.VMEM((3, M, HALF), BF),
            pltpu.VMEM((3, M, HALF), BF),
            pltpu.VMEM((M, HALF), BF),
            pltpu.VMEM((M, HALF), BF),
            pltpu.VMEM((2, M, HALF), F32),
            pltpu.SemaphoreType.DMA((3, NB)),
            pltpu.SemaphoreType.DMA((3, NB)),
            pltpu.SemaphoreType.DMA((3, NB)),
            pltpu.SemaphoreType.DMA((3, NB)),
            pltpu.SemaphoreType.DMA((8,)),
        ],
        compiler_params=pltpu.CompilerParams(
            collective_id=0,
            vmem_limit_bytes=62 * 1024 * 1024,
        ),
    )(x)


# device time: 120315 ns/iter; 2.7047x vs baseline; 2.4134x over previous
import jax
import jax.numpy as jnp
from jax import lax
from jax.experimental import pallas as pl
from jax.experimental.pallas import tpu as pltpu

NZ = 4
M = 4096
N = 4096
CHUNK = N // NZ
QW = CHUNK // 4
W = QW // 2
NB = 2
RB = M // NB

BF = jnp.bfloat16
F32 = jnp.float32


def kernel(x):

    def body(x_ref, out_ref, comm_r, comm_l, s0_r, s0_l, fa,
             ss_r, rs_r, ss_l, rs_l,
             ag_ss_x, ag_rs_x, ag_ss_y, ag_rs_y, local_sems):
        xi = lax.axis_index("x")
        yi = lax.axis_index("y")
        zi = lax.axis_index("z")
        right = (zi + 1) % NZ
        left = (zi - 1) % NZ
        q = 2 * xi + yi
        qx = 2 * (1 - xi) + yi
        qy = 2 * xi + (1 - yi)
        qd = 2 * (1 - xi) + (1 - yi)

        r_c0 = (zi + 3) % NZ
        r_adds = [(zi + 2) % NZ, (zi + 1) % NZ, zi]
        l_c0 = (zi + 1) % NZ
        l_adds = [(zi + 2) % NZ, (zi + 3) % NZ, zi]

        def lo(c):
            return pl.ds(c * CHUNK + q * QW, W)

        def hi(c):
            return pl.ds(c * CHUNK + q * QW + W, W)

        def load(col_slice, dst, sem_i):
            cp = pltpu.make_async_copy(
                x_ref.at[0, :, col_slice], dst, local_sems.at[sem_i])
            cp.start()
            return cp

        def rb(b):
            return slice(b * RB, (b + 1) * RB)

        def hop(comm, ss, rs, dev, s, b, src):
            return pltpu.make_async_remote_copy(
                src_ref=src,
                dst_ref=comm.at[s, pl.ds(b * RB, RB), :],
                send_sem=ss.at[s, b], recv_sem=rs.at[s, b],
                device_id=(xi, yi, dev),
                device_id_type=pl.DeviceIdType.MESH)

        def ag(dev_id, quarter, b, ss, rs):
            sl = (pl.ds(b * RB, RB), pl.ds(quarter * QW, QW))
            return pltpu.make_async_remote_copy(
                src_ref=out_ref.at[sl[0], sl[1]],
                dst_ref=out_ref.at[sl[0], sl[1]],
                send_sem=ss, recv_sem=rs,
                device_id=dev_id,
                device_id_type=pl.DeviceIdType.MESH)

        ld_r = load(lo(r_c0), fa.at[0], 0)
        ld_l = load(hi(l_c0), fa.at[1], 1)

        barrier_sem = pltpu.get_barrier_semaphore()
        for dev in ((xi, yi, left), (xi, yi, right),
                    (1 - xi, yi, zi), (xi, 1 - yi, zi)):
            pl.semaphore_signal(
                barrier_sem, inc=1,
                device_id=dev, device_id_type=pl.DeviceIdType.MESH)
        pl.semaphore_wait(barrier_sem, 4)

        sends = []

        ld_r.wait()
        ld_l.wait()
        for b in range(NB):
            s0_r[rb(b), :] = fa[0, rb(b), :].astype(BF)
            d = hop(comm_r, ss_r, rs_r, right, 0, b,
                    s0_r.at[pl.ds(b * RB, RB), :])
            d.start()
            sends.append(d)
            s0_l[rb(b), :] = fa[1, rb(b), :].astype(BF)
            d = hop(comm_l, ss_l, rs_l, left, 0, b,
                    s0_l.at[pl.ds(b * RB, RB), :])
            d.start()
            sends.append(d)

        ld_r = load(lo(r_adds[0]), fa.at[0], 2)
        ld_l = load(hi(l_adds[0]), fa.at[1], 3)

        for s in range(3):
            ld_r.wait()
            ld_l.wait()
            for b in range(NB):
                hop(comm_r, ss_r, rs_r, right, s, b,
                    s0_r.at[pl.ds(b * RB, RB), :]).wait_recv()
                hop(comm_l, ss_l, rs_l, left, s, b,
                    s0_l.at[pl.ds(b * RB, RB), :]).wait_recv()
                if s < 2:
                    comm_r[s, rb(b), :] = (
                        fa[0, rb(b), :]
                        + comm_r[s, rb(b), :].astype(F32)).astype(BF)
                    d = hop(comm_r, ss_r, rs_r, right, s + 1, b,
                            comm_r.at[s, pl.ds(b * RB, RB), :])
                    d.start()
                    sends.append(d)
                    comm_l[s, rb(b), :] = (
                        fa[1, rb(b), :]
                        + comm_l[s, rb(b), :].astype(F32)).astype(BF)
                    d = hop(comm_l, ss_l, rs_l, left, s + 1, b,
                            comm_l.at[s, pl.ds(b * RB, RB), :])
                    d.start()
                    sends.append(d)
                else:
                    out_ref[rb(b), pl.ds(q * QW, W)] = (
                        fa[0, rb(b), :]
                        + comm_r[2, rb(b), :].astype(F32)).astype(BF)
                    out_ref[rb(b), pl.ds(q * QW + W, W)] = (
                        fa[1, rb(b), :]
                        + comm_l[2, rb(b), :].astype(F32)).astype(BF)
                    d = ag((1 - xi, yi, zi), q, b,
                           ag_ss_x.at[0, b], ag_rs_x.at[0, b])
                    d.start()
                    sends.append(d)
                    d = ag((xi, 1 - yi, zi), q, b,
                           ag_ss_y.at[0, b], ag_rs_y.at[0, b])
                    d.start()
                    sends.append(d)
            if s < 2:
                ld_r = load(lo(r_adds[s + 1]), fa.at[0], 4 + 2 * s)
                ld_l = load(hi(l_adds[s + 1]), fa.at[1], 5 + 2 * s)

        for b in range(NB):
            ag((1 - xi, yi, zi), qx, b,
               ag_ss_x.at[0, b], ag_rs_x.at[0, b]).wait_recv()
            d = ag((xi, 1 - yi, zi), qx, b,
                   ag_ss_y.at[1, b], ag_rs_y.at[1, b])
            d.start()
            sends.append(d)
        for b in range(NB):
            ag((xi, 1 - yi, zi), qy, b,
               ag_ss_y.at[0, b], ag_rs_y.at[0, b]).wait_recv()
        for b in range(NB):
            ag((xi, 1 - yi, zi), qd, b,
               ag_ss_y.at[1, b], ag_rs_y.at[1, b]).wait_recv()

        for d in sends:
            d.wait_send()

    return pl.pallas_call(
        body,
        out_shape=jax.ShapeDtypeStruct((M, CHUNK), BF),
        in_specs=[pl.BlockSpec(memory_space=pltpu.MemorySpace.HBM)],
        out_specs=pl.BlockSpec(memory_space=pltpu.VMEM),
        scratch_shapes=[
            pltpu.VMEM((3, M, W), BF),
            pltpu.VMEM((3, M, W), BF),
            pltpu.VMEM((M, W), BF),
            pltpu.VMEM((M, W), BF),
            pltpu.VMEM((2, M, W), F32),
            pltpu.SemaphoreType.DMA((3, NB)),
            pltpu.SemaphoreType.DMA((3, NB)),
            pltpu.SemaphoreType.DMA((3, NB)),
            pltpu.SemaphoreType.DMA((3, NB)),
            pltpu.SemaphoreType.DMA((1, NB)),
            pltpu.SemaphoreType.DMA((1, NB)),
            pltpu.SemaphoreType.DMA((2, NB)),
            pltpu.SemaphoreType.DMA((2, NB)),
            pltpu.SemaphoreType.DMA((8,)),
        ],
        compiler_params=pltpu.CompilerParams(
            collective_id=0,
            vmem_limit_bytes=62 * 1024 * 1024,
        ),
    )(x)


# device time: 110079 ns/iter; 2.9562x vs baseline; 1.0930x over previous
import jax
import jax.numpy as jnp
from jax import lax
from jax.experimental import pallas as pl
from jax.experimental.pallas import tpu as pltpu

NZ = 4
M = 4096
N = 4096
CHUNK = N // NZ
QW = CHUNK // 4
W = QW // 2
NB = 2
RB = M // NB

BF = jnp.bfloat16
F32 = jnp.float32


def kernel(x):

    def body(x_ref, out_ref, comm_r, comm_l, s0_r, s0_l, fa,
             ss_r, rs_r, ss_l, rs_l,
             ag_ss_x, ag_rs_x, ag_ss_y, ag_rs_y, local_sems):
        xi = lax.axis_index("x")
        yi = lax.axis_index("y")
        zi = lax.axis_index("z")
        right = (zi + 1) % NZ
        left = (zi - 1) % NZ
        q = 2 * xi + yi
        qx = 2 * (1 - xi) + yi
        qy = 2 * xi + (1 - yi)
        qd = 2 * (1 - xi) + (1 - yi)

        r_c0 = (zi + 3) % NZ
        r_adds = [(zi + 2) % NZ, (zi + 1) % NZ, zi]
        l_c0 = (zi + 1) % NZ
        l_adds = [(zi + 2) % NZ, (zi + 3) % NZ, zi]

        def lo(c):
            return pl.ds(c * CHUNK + q * QW, W)

        def hi(c):
            return pl.ds(c * CHUNK + q * QW + W, W)

        def load(col_slice, dst, sem_i):
            cp = pltpu.make_async_copy(
                x_ref.at[0, :, col_slice], dst, local_sems.at[sem_i])
            cp.start()
            return cp

        def rb(b):
            return slice(b * RB, (b + 1) * RB)

        def hop(comm, ss, rs, dev, s, b, src):
            return pltpu.make_async_remote_copy(
                src_ref=src,
                dst_ref=comm.at[s, pl.ds(b * RB, RB), :],
                send_sem=ss.at[s, b], recv_sem=rs.at[s, b],
                device_id=(xi, yi, dev),
                device_id_type=pl.DeviceIdType.MESH)

        def ag(dev_id, col, width, b, ss, rs):
            sl = (pl.ds(b * RB, RB), pl.ds(col, width))
            return pltpu.make_async_remote_copy(
                src_ref=out_ref.at[sl[0], sl[1]],
                dst_ref=out_ref.at[sl[0], sl[1]],
                send_sem=ss, recv_sem=rs,
                device_id=dev_id,
                device_id_type=pl.DeviceIdType.MESH)

        ld_r = load(lo(r_c0), fa.at[0], 0)
        ld_l = load(hi(l_c0), fa.at[1], 1)

        barrier_sem = pltpu.get_barrier_semaphore()
        for dev in ((xi, yi, left), (xi, yi, right),
                    (1 - xi, yi, zi), (xi, 1 - yi, zi)):
            pl.semaphore_signal(
                barrier_sem, inc=1,
                device_id=dev, device_id_type=pl.DeviceIdType.MESH)
        pl.semaphore_wait(barrier_sem, 4)

        sends = []

        ld_r.wait()
        ld_l.wait()
        for b in range(NB):
            s0_r[rb(b), :] = fa[0, rb(b), :].astype(BF)
            d = hop(comm_r, ss_r, rs_r, right, 0, b,
                    s0_r.at[pl.ds(b * RB, RB), :])
            d.start()
            sends.append(d)
            s0_l[rb(b), :] = fa[1, rb(b), :].astype(BF)
            d = hop(comm_l, ss_l, rs_l, left, 0, b,
                    s0_l.at[pl.ds(b * RB, RB), :])
            d.start()
            sends.append(d)

        ld_r = load(lo(r_adds[0]), fa.at[0], 2)
        ld_l = load(hi(l_adds[0]), fa.at[1], 3)

        for s in range(3):
            ld_r.wait()
            ld_l.wait()
            for b in range(NB):
                hop(comm_r, ss_r, rs_r, right, s, b,
                    s0_r.at[pl.ds(b * RB, RB), :]).wait_recv()
                hop(comm_l, ss_l, rs_l, left, s, b,
                    s0_l.at[pl.ds(b * RB, RB), :]).wait_recv()
                if s < 2:
                    comm_r[s, rb(b), :] = (
                        fa[0, rb(b), :]
                        + comm_r[s, rb(b), :].astype(F32)).astype(BF)
                    d = hop(comm_r, ss_r, rs_r, right, s + 1, b,
                            comm_r.at[s, pl.ds(b * RB, RB), :])
                    d.start()
                    sends.append(d)
                    comm_l[s, rb(b), :] = (
                        fa[1, rb(b), :]
                        + comm_l[s, rb(b), :].astype(F32)).astype(BF)
                    d = hop(comm_l, ss_l, rs_l, left, s + 1, b,
                            comm_l.at[s, pl.ds(b * RB, RB), :])
                    d.start()
                    sends.append(d)
                else:
                    out_ref[rb(b), pl.ds(q * QW, W)] = (
                        fa[0, rb(b), :]
                        + comm_r[2, rb(b), :].astype(F32)).astype(BF)
                    out_ref[rb(b), pl.ds(q * QW + W, W)] = (
                        fa[1, rb(b), :]
                        + comm_l[2, rb(b), :].astype(F32)).astype(BF)
                    d = ag((1 - xi, yi, zi), q * QW, QW, b,
                           ag_ss_x.at[0, b], ag_rs_x.at[0, b])
                    d.start()
                    sends.append(d)
                    d = ag((xi, 1 - yi, zi), q * QW, QW, b,
                           ag_ss_y.at[0, b], ag_rs_y.at[0, b])
                    d.start()
                    sends.append(d)
            if s < 2:
                ld_r = load(lo(r_adds[s + 1]), fa.at[0], 4 + 2 * s)
                ld_l = load(hi(l_adds[s + 1]), fa.at[1], 5 + 2 * s)

        for b in range(NB):
            ag((1 - xi, yi, zi), qx * QW, QW, b,
               ag_ss_x.at[0, b], ag_rs_x.at[0, b]).wait_recv()
            d = ag((xi, 1 - yi, zi), qx * QW, W, b,
                   ag_ss_y.at[1, b], ag_rs_y.at[1, b])
            d.start()
            sends.append(d)
        for b in range(NB):
            ag((xi, 1 - yi, zi), qy * QW, QW, b,
               ag_ss_y.at[0, b], ag_rs_y.at[0, b]).wait_recv()
            d = ag((1 - xi, yi, zi), qy * QW + W, W, b,
                   ag_ss_x.at[1, b], ag_rs_x.at[1, b])
            d.start()
            sends.append(d)
        for b in range(NB):
            ag((xi, 1 - yi, zi), qd * QW, W, b,
               ag_ss_y.at[1, b], ag_rs_y.at[1, b]).wait_recv()
            ag((1 - xi, yi, zi), qd * QW + W, W, b,
               ag_ss_x.at[1, b], ag_rs_x.at[1, b]).wait_recv()

        for d in sends:
            d.wait_send()

    return pl.pallas_call(
        body,
        out_shape=jax.ShapeDtypeStruct((M, CHUNK), BF),
        in_specs=[pl.BlockSpec(memory_space=pltpu.MemorySpace.HBM)],
        out_specs=pl.BlockSpec(memory_space=pltpu.VMEM),
        scratch_shapes=[
            pltpu.VMEM((3, M, W), BF),
            pltpu.VMEM((3, M, W), BF),
            pltpu.VMEM((M, W), BF),
            pltpu.VMEM((M, W), BF),
            pltpu.VMEM((2, M, W), F32),
            pltpu.SemaphoreType.DMA((3, NB)),
            pltpu.SemaphoreType.DMA((3, NB)),
            pltpu.SemaphoreType.DMA((3, NB)),
            pltpu.SemaphoreType.DMA((3, NB)),
            pltpu.SemaphoreType.DMA((2, NB)),
            pltpu.SemaphoreType.DMA((2, NB)),
            pltpu.SemaphoreType.DMA((2, NB)),
            pltpu.SemaphoreType.DMA((2, NB)),
            pltpu.SemaphoreType.DMA((8,)),
        ],
        compiler_params=pltpu.CompilerParams(
            collective_id=0,
            vmem_limit_bytes=62 * 1024 * 1024,
        ),
    )(x)


# device time: 104689 ns/iter; 3.1084x vs baseline; 1.0515x over previous
import jax
import jax.numpy as jnp
from jax import lax
from jax.experimental import pallas as pl
from jax.experimental.pallas import tpu as pltpu

NZ = 4
M = 4096
N = 4096
CHUNK = N // NZ
QW = CHUNK // 4
W = QW // 2
NB = 4
RB = M // NB

BF = jnp.bfloat16
F32 = jnp.float32


def kernel(x):

    def body(x_ref, out_ref, comm_r, comm_l, s0_r, s0_l, fa,
             ss_r, rs_r, ss_l, rs_l,
             ag_ss_x, ag_rs_x, ag_ss_y, ag_rs_y, local_sems):
        xi = lax.axis_index("x")
        yi = lax.axis_index("y")
        zi = lax.axis_index("z")
        right = (zi + 1) % NZ
        left = (zi - 1) % NZ
        q = 2 * xi + yi
        qx = 2 * (1 - xi) + yi
        qy = 2 * xi + (1 - yi)
        qd = 2 * (1 - xi) + (1 - yi)

        r_c0 = (zi + 3) % NZ
        r_adds = [(zi + 2) % NZ, (zi + 1) % NZ, zi]
        l_c0 = (zi + 1) % NZ
        l_adds = [(zi + 2) % NZ, (zi + 3) % NZ, zi]

        def lo(c):
            return pl.ds(c * CHUNK + q * QW, W)

        def hi(c):
            return pl.ds(c * CHUNK + q * QW + W, W)

        def load(col_slice, dst, sem_i):
            cp = pltpu.make_async_copy(
                x_ref.at[0, :, col_slice], dst, local_sems.at[sem_i])
            cp.start()
            return cp

        def rb(b):
            return slice(b * RB, (b + 1) * RB)

        def hop(comm, ss, rs, dev, s, b, src):
            return pltpu.make_async_remote_copy(
                src_ref=src,
                dst_ref=comm.at[s, pl.ds(b * RB, RB), :],
                send_sem=ss.at[s, b], recv_sem=rs.at[s, b],
                device_id=(xi, yi, dev),
                device_id_type=pl.DeviceIdType.MESH)

        def ag(dev_id, col, width, b, ss, rs):
            sl = (pl.ds(b * RB, RB), pl.ds(col, width))
            return pltpu.make_async_remote_copy(
                src_ref=out_ref.at[sl[0], sl[1]],
                dst_ref=out_ref.at[sl[0], sl[1]],
                send_sem=ss, recv_sem=rs,
                device_id=dev_id,
                device_id_type=pl.DeviceIdType.MESH)

        ld_r = load(lo(r_c0), fa.at[0], 0)
        ld_l = load(hi(l_c0), fa.at[1], 1)

        barrier_sem = pltpu.get_barrier_semaphore()
        for dev in ((xi, yi, left), (xi, yi, right),
                    (1 - xi, yi, zi), (xi, 1 - yi, zi)):
            pl.semaphore_signal(
                barrier_sem, inc=1,
                device_id=dev, device_id_type=pl.DeviceIdType.MESH)
        pl.semaphore_wait(barrier_sem, 4)

        sends = []

        ld_r.wait()
        ld_l.wait()
        for b in range(NB):
            s0_r[rb(b), :] = fa[0, rb(b), :].astype(BF)
            d = hop(comm_r, ss_r, rs_r, right, 0, b,
                    s0_r.at[pl.ds(b * RB, RB), :])
            d.start()
            sends.append(d)
            s0_l[rb(b), :] = fa[1, rb(b), :].astype(BF)
            d = hop(comm_l, ss_l, rs_l, left, 0, b,
                    s0_l.at[pl.ds(b * RB, RB), :])
            d.start()
            sends.append(d)

        ld_r = load(lo(r_adds[0]), fa.at[0], 2)
        ld_l = load(hi(l_adds[0]), fa.at[1], 3)

        for s in range(3):
            ld_r.wait()
            ld_l.wait()
            for b in range(NB):
                hop(comm_r, ss_r, rs_r, right, s, b,
                    s0_r.at[pl.ds(b * RB, RB), :]).wait_recv()
                hop(comm_l, ss_l, rs_l, left, s, b,
                    s0_l.at[pl.ds(b * RB, RB), :]).wait_recv()
                if s < 2:
                    comm_r[s, rb(b), :] = (
                        fa[0, rb(b), :]
                        + comm_r[s, rb(b), :].astype(F32)).astype(BF)
                    d = hop(comm_r, ss_r, rs_r, right, s + 1, b,
                            comm_r.at[s, pl.ds(b * RB, RB), :])
                    d.start()
                    sends.append(d)
                    comm_l[s, rb(b), :] = (
                        fa[1, rb(b), :]
                        + comm_l[s, rb(b), :].astype(F32)).astype(BF)
                    d = hop(comm_l, ss_l, rs_l, left, s + 1, b,
                            comm_l.at[s, pl.ds(b * RB, RB), :])
                    d.start()
                    sends.append(d)
                else:
                    out_ref[rb(b), pl.ds(q * QW, W)] = (
                        fa[0, rb(b), :]
                        + comm_r[2, rb(b), :].astype(F32)).astype(BF)
                    out_ref[rb(b), pl.ds(q * QW + W, W)] = (
                        fa[1, rb(b), :]
                        + comm_l[2, rb(b), :].astype(F32)).astype(BF)
                    d = ag((1 - xi, yi, zi), q * QW, QW, b,
                           ag_ss_x.at[0, b], ag_rs_x.at[0, b])
                    d.start()
                    sends.append(d)
                    d = ag((xi, 1 - yi, zi), q * QW, QW, b,
                           ag_ss_y.at[0, b], ag_rs_y.at[0, b])
                    d.start()
                    sends.append(d)
            if s < 2:
                ld_r = load(lo(r_adds[s + 1]), fa.at[0], 4 + 2 * s)
                ld_l = load(hi(l_adds[s + 1]), fa.at[1], 5 + 2 * s)

        for b in range(NB):
            ag((1 - xi, yi, zi), qx * QW, QW, b,
               ag_ss_x.at[0, b], ag_rs_x.at[0, b]).wait_recv()
            d = ag((xi, 1 - yi, zi), qx * QW, W, b,
                   ag_ss_y.at[1, b], ag_rs_y.at[1, b])
            d.start()
            sends.append(d)
        for b in range(NB):
            ag((xi, 1 - yi, zi), qy * QW, QW, b,
               ag_ss_y.at[0, b], ag_rs_y.at[0, b]).wait_recv()
            d = ag((1 - xi, yi, zi), qy * QW + W, W, b,
                   ag_ss_x.at[1, b], ag_rs_x.at[1, b])
            d.start()
            sends.append(d)
        for b in range(NB):
            ag((xi, 1 - yi, zi), qd * QW, W, b,
               ag_ss_y.at[1, b], ag_rs_y.at[1, b]).wait_recv()
            ag((1 - xi, yi, zi), qd * QW + W, W, b,
               ag_ss_x.at[1, b], ag_rs_x.at[1, b]).wait_recv()

        for d in sends:
            d.wait_send()

    return pl.pallas_call(
        body,
        out_shape=jax.ShapeDtypeStruct((M, CHUNK), BF),
        in_specs=[pl.BlockSpec(memory_space=pltpu.MemorySpace.HBM)],
        out_specs=pl.BlockSpec(memory_space=pltpu.VMEM),
        scratch_shapes=[
            pltpu.VMEM((3, M, W), BF),
            pltpu.VMEM((3, M, W), BF),
            pltpu.VMEM((M, W), BF),
            pltpu.VMEM((M, W), BF),
            pltpu.VMEM((2, M, W), F32),
            pltpu.SemaphoreType.DMA((3, NB)),
            pltpu.SemaphoreType.DMA((3, NB)),
            pltpu.SemaphoreType.DMA((3, NB)),
            pltpu.SemaphoreType.DMA((3, NB)),
            pltpu.SemaphoreType.DMA((2, NB)),
            pltpu.SemaphoreType.DMA((2, NB)),
            pltpu.SemaphoreType.DMA((2, NB)),
            pltpu.SemaphoreType.DMA((2, NB)),
            pltpu.SemaphoreType.DMA((8,)),
        ],
        compiler_params=pltpu.CompilerParams(
            collective_id=0,
            vmem_limit_bytes=62 * 1024 * 1024,
        ),
    )(x)


# device time: 103033 ns/iter; 3.1584x vs baseline; 1.0161x over previous
import jax
import jax.numpy as jnp
from jax import lax
from jax.experimental import pallas as pl
from jax.experimental.pallas import tpu as pltpu

NZ = 4
M = 4096
N = 4096
CHUNK = N // NZ
QW = CHUNK // 4
W = QW // 2
NB = 8
RB = M // NB

BF = jnp.bfloat16
F32 = jnp.float32


def kernel(x):

    def body(x_ref, out_ref, comm_r, comm_l, s0_r, s0_l, fa,
             ss_r, rs_r, ss_l, rs_l,
             ag_ss_x, ag_rs_x, ag_ss_y, ag_rs_y, local_sems):
        xi = lax.axis_index("x")
        yi = lax.axis_index("y")
        zi = lax.axis_index("z")
        right = (zi + 1) % NZ
        left = (zi - 1) % NZ
        q = 2 * xi + yi
        qx = 2 * (1 - xi) + yi
        qy = 2 * xi + (1 - yi)
        qd = 2 * (1 - xi) + (1 - yi)

        r_c0 = (zi + 3) % NZ
        r_adds = [(zi + 2) % NZ, (zi + 1) % NZ, zi]
        l_c0 = (zi + 1) % NZ
        l_adds = [(zi + 2) % NZ, (zi + 3) % NZ, zi]

        def lo(c):
            return pl.ds(c * CHUNK + q * QW, W)

        def hi(c):
            return pl.ds(c * CHUNK + q * QW + W, W)

        def load(col_slice, dst, sem_i):
            cp = pltpu.make_async_copy(
                x_ref.at[0, :, col_slice], dst, local_sems.at[sem_i])
            cp.start()
            return cp

        def rb(b):
            return slice(b * RB, (b + 1) * RB)

        def hop(comm, ss, rs, dev, s, b, src):
            return pltpu.make_async_remote_copy(
                src_ref=src,
                dst_ref=comm.at[s, pl.ds(b * RB, RB), :],
                send_sem=ss.at[s, b], recv_sem=rs.at[s, b],
                device_id=(xi, yi, dev),
                device_id_type=pl.DeviceIdType.MESH)

        def ag(dev_id, col, width, b, ss, rs):
            sl = (pl.ds(b * RB, RB), pl.ds(col, width))
            return pltpu.make_async_remote_copy(
                src_ref=out_ref.at[sl[0], sl[1]],
                dst_ref=out_ref.at[sl[0], sl[1]],
                send_sem=ss, recv_sem=rs,
                device_id=dev_id,
                device_id_type=pl.DeviceIdType.MESH)

        ld_r = load(lo(r_c0), fa.at[0], 0)
        ld_l = load(hi(l_c0), fa.at[1], 1)

        barrier_sem = pltpu.get_barrier_semaphore()
        for dev in ((xi, yi, left), (xi, yi, right),
                    (1 - xi, yi, zi), (xi, 1 - yi, zi)):
            pl.semaphore_signal(
                barrier_sem, inc=1,
                device_id=dev, device_id_type=pl.DeviceIdType.MESH)
        pl.semaphore_wait(barrier_sem, 4)

        sends = []

        ld_r.wait()
        ld_l.wait()
        for b in range(NB):
            s0_r[rb(b), :] = fa[0, rb(b), :].astype(BF)
            d = hop(comm_r, ss_r, rs_r, right, 0, b,
                    s0_r.at[pl.ds(b * RB, RB), :])
            d.start()
            sends.append(d)
            s0_l[rb(b), :] = fa[1, rb(b), :].astype(BF)
            d = hop(comm_l, ss_l, rs_l, left, 0, b,
                    s0_l.at[pl.ds(b * RB, RB), :])
            d.start()
            sends.append(d)

        ld_r = load(lo(r_adds[0]), fa.at[0], 2)
        ld_l = load(hi(l_adds[0]), fa.at[1], 3)

        for s in range(3):
            ld_r.wait()
            ld_l.wait()
            for b in range(NB):
                hop(comm_r, ss_r, rs_r, right, s, b,
                    s0_r.at[pl.ds(b * RB, RB), :]).wait_recv()
                hop(comm_l, ss_l, rs_l, left, s, b,
                    s0_l.at[pl.ds(b * RB, RB), :]).wait_recv()
                if s < 2:
                    comm_r[s, rb(b), :] = (
                        fa[0, rb(b), :]
                        + comm_r[s, rb(b), :].astype(F32)).astype(BF)
                    d = hop(comm_r, ss_r, rs_r, right, s + 1, b,
                            comm_r.at[s, pl.ds(b * RB, RB), :])
                    d.start()
                    sends.append(d)
                    comm_l[s, rb(b), :] = (
                        fa[1, rb(b), :]
                        + comm_l[s, rb(b), :].astype(F32)).astype(BF)
                    d = hop(comm_l, ss_l, rs_l, left, s + 1, b,
                            comm_l.at[s, pl.ds(b * RB, RB), :])
                    d.start()
                    sends.append(d)
                else:
                    out_ref[rb(b), pl.ds(q * QW, W)] = (
                        fa[0, rb(b), :]
                        + comm_r[2, rb(b), :].astype(F32)).astype(BF)
                    out_ref[rb(b), pl.ds(q * QW + W, W)] = (
                        fa[1, rb(b), :]
                        + comm_l[2, rb(b), :].astype(F32)).astype(BF)
                    d = ag((1 - xi, yi, zi), q * QW, QW, b,
                           ag_ss_x.at[0, b], ag_rs_x.at[0, b])
                    d.start()
                    sends.append(d)
                    d = ag((xi, 1 - yi, zi), q * QW, QW, b,
                           ag_ss_y.at[0, b], ag_rs_y.at[0, b])
                    d.start()
                    sends.append(d)
            if s < 2:
                ld_r = load(lo(r_adds[s + 1]), fa.at[0], 4 + 2 * s)
                ld_l = load(hi(l_adds[s + 1]), fa.at[1], 5 + 2 * s)

        for b in range(NB):
            ag((1 - xi, yi, zi), qx * QW, QW, b,
               ag_ss_x.at[0, b], ag_rs_x.at[0, b]).wait_recv()
            d = ag((xi, 1 - yi, zi), qx * QW, W, b,
                   ag_ss_y.at[1, b], ag_rs_y.at[1, b])
            d.start()
            sends.append(d)
        for b in range(NB):
            ag((xi, 1 - yi, zi), qy * QW, QW, b,
               ag_ss_y.at[0, b], ag_rs_y.at[0, b]).wait_recv()
            d = ag((1 - xi, yi, zi), qy * QW + W, W, b,
                   ag_ss_x.at[1, b], ag_rs_x.at[1, b])
            d.start()
            sends.append(d)
        for b in range(NB):
            ag((xi, 1 - yi, zi), qd * QW, W, b,
               ag_ss_y.at[1, b], ag_rs_y.at[1, b]).wait_recv()
            ag((1 - xi, yi, zi), qd * QW + W, W, b,
               ag_ss_x.at[1, b], ag_rs_x.at[1, b]).wait_recv()

        for d in sends:
            d.wait_send()

    return pl.pallas_call(
        body,
        out_shape=jax.ShapeDtypeStruct((M, CHUNK), BF),
        in_specs=[pl.BlockSpec(memory_space=pltpu.MemorySpace.HBM)],
        out_specs=pl.BlockSpec(memory_space=pltpu.VMEM),
        scratch_shapes=[
            pltpu.VMEM((3, M, W), BF),
            pltpu.VMEM((3, M, W), BF),
            pltpu.VMEM((M, W), BF),
            pltpu.VMEM((M, W), BF),
            pltpu.VMEM((2, M, W), F32),
            pltpu.SemaphoreType.DMA((3, NB)),
            pltpu.SemaphoreType.DMA((3, NB)),
            pltpu.SemaphoreType.DMA((3, NB)),
            pltpu.SemaphoreType.DMA((3, NB)),
            pltpu.SemaphoreType.DMA((2, NB)),
            pltpu.SemaphoreType.DMA((2, NB)),
            pltpu.SemaphoreType.DMA((2, NB)),
            pltpu.SemaphoreType.DMA((2, NB)),
            pltpu.SemaphoreType.DMA((8,)),
        ],
        compiler_params=pltpu.CompilerParams(
            collective_id=0,
            vmem_limit_bytes=62 * 1024 * 1024,
        ),
    )(x)
